# Initial kernel scaffold; baseline (speedup 1.0000x reference)
import jax
import jax.numpy as jnp
from jax.experimental import pallas as pl


def kernel(input, fake_param):
    raise NotImplementedError("write your pallas kernel here")


# speedup vs baseline: 11.5370x; 11.5370x over previous
import jax
import jax.numpy as jnp
from jax import lax
from jax.experimental import pallas as pl
from jax.experimental.pallas import tpu as pltpu
from jax.experimental.pallas import tpu_sc as plsc

ROWS = 128
COLS = 32768
FILLER = 32

NC = 2
NS = 16
LANES = 16
RB = 8
N_BLOCKS = ROWS // RB

SCAN_CH = 512
N_SCAN_CH = COLS // SCAN_CH
FILL_CH = 4096
THRESH = 0.5


def _sc_body(in_hbm, fp_hbm, out_hbm, in_buf, fp_buf, out_buf, zero_buf, sem):
    c = lax.axis_index("c")
    s = lax.axis_index("s")
    blk = c * (N_BLOCKS // NC) + s
    row0 = blk * RB

    @pl.when(s < N_BLOCKS // NC)
    def _owner():
        for rr in range(RB):
            def zero_init(i, carry, rr=rr):
                zero_buf[rr, pl.ds(i * LANES, LANES)] = jnp.zeros(
                    (LANES,), jnp.float32)
                return carry
            lax.fori_loop(0, FILL_CH // LANES, zero_init, 0)

        def scan_cond(state):
            ch = state[0]
            cnts = state[1:]
            cnt_min = cnts[0]
            for v in cnts[1:]:
                cnt_min = jnp.minimum(cnt_min, v)
            return jnp.logical_and(cnt_min < FILLER, ch < N_SCAN_CH)

        def scan_body(state):
            ch = state[0]
            cnts = list(state[1:])
            start = pl.multiple_of(ch * SCAN_CH, SCAN_CH)
            pltpu.async_copy(
                in_hbm.at[pl.ds(row0, RB), pl.ds(start, SCAN_CH)],
                in_buf, sem).wait()
            pltpu.async_copy(
                fp_hbm.at[pl.ds(row0, RB), pl.ds(start, SCAN_CH)],
                fp_buf, sem).wait()
            for rr in range(RB):
                def vec_body(i, cnt, rr=rr):
                    vi = in_buf[rr, pl.ds(i * LANES, LANES)]
                    vf = fp_buf[rr, pl.ds(i * LANES, LANES)]
                    m = jnp.logical_and(vi > 0.0, vf > THRESH)
                    ones = jnp.where(m, jnp.float32(1.0), jnp.float32(0.0))
                    cs = plsc.cumsum(ones)
                    keep = jnp.logical_and(
                        m, (cnt.astype(jnp.float32) + cs)
                        <= jnp.float32(FILLER))
                    out_buf[rr, pl.ds(i * LANES, LANES)] = jnp.where(
                        keep, jnp.float32(1.0), jnp.float32(0.0))
                    return cnt + jnp.sum(ones).astype(jnp.int32)
                cnts[rr] = lax.fori_loop(
                    0, SCAN_CH // LANES, vec_body, cnts[rr])
            pltpu.async_copy(
                out_buf, out_hbm.at[pl.ds(row0, RB), pl.ds(start, SCAN_CH)],
                sem).wait()
            return (ch + 1, *cnts)

        end_state = lax.while_loop(
            scan_cond, scan_body, (0,) + (0,) * RB)
        ch_end = end_state[0]

        def align_cond(start):
            return jnp.logical_and(start < COLS, start % FILL_CH != 0)

        def align_body(start):
            pltpu.async_copy(
                zero_buf.at[slice(None), pl.ds(0, SCAN_CH)],
                out_hbm.at[pl.ds(row0, RB),
                           pl.ds(pl.multiple_of(start, SCAN_CH), SCAN_CH)],
                sem).wait()
            return start + SCAN_CH

        start = lax.while_loop(align_cond, align_body, ch_end * SCAN_CH)

        def fill_cond(start):
            return start < COLS

        def fill_body(start):
            pltpu.async_copy(
                zero_buf,
                out_hbm.at[pl.ds(row0, RB),
                           pl.ds(pl.multiple_of(start, FILL_CH), FILL_CH)],
                sem).wait()
            return start + FILL_CH

        lax.while_loop(fill_cond, fill_body, start)


@jax.jit
def _fake_profile_sc(inp, fp):
    mesh = plsc.VectorSubcoreMesh(
        core_axis_name="c", subcore_axis_name="s",
        num_cores=NC, num_subcores=NS)
    return pl.kernel(
        _sc_body,
        out_type=jax.ShapeDtypeStruct((ROWS, COLS), jnp.float32),
        mesh=mesh,
        scratch_types=[
            pltpu.VMEM((RB, SCAN_CH), jnp.float32),
            pltpu.VMEM((RB, SCAN_CH), jnp.float32),
            pltpu.VMEM((RB, SCAN_CH), jnp.float32),
            pltpu.VMEM((RB, FILL_CH), jnp.float32),
            pltpu.SemaphoreType.DMA,
        ],
        compiler_params=pltpu.CompilerParams(needs_layout_passes=False),
    )(inp, fp)


def kernel(input, fake_param):
    return _fake_profile_sc(input, fake_param)


# speedup vs baseline: 14.2257x; 1.2331x over previous
import jax
import jax.numpy as jnp
from jax import lax
from jax.experimental import pallas as pl
from jax.experimental.pallas import tpu as pltpu
from jax.experimental.pallas import tpu_sc as plsc

ROWS = 128
COLS = 32768
FILLER = 32

NC = 2
NS = 16
LANES = 16
RB = 8
N_BLOCKS = ROWS // RB

SCAN_CH = 512
N_SCAN_CH = COLS // SCAN_CH
FILL_CH = 1024
N_FILL = COLS // FILL_CH
THRESH = 0.5


def _sc_body(in_hbm, fp_hbm, out_hbm, in_buf, fp_buf, out_buf, zero_buf,
             sem_in, sem_out, sem_fill):
    c = lax.axis_index("c")
    s = lax.axis_index("s")
    blk = c * (N_BLOCKS // NC) + s
    row0 = blk * RB

    @pl.when(s < N_BLOCKS // NC)
    def _owner():
        pltpu.async_copy(
            in_hbm.at[pl.ds(row0, RB), pl.ds(0, SCAN_CH)], in_buf, sem_in)
        pltpu.async_copy(
            fp_hbm.at[pl.ds(row0, RB), pl.ds(0, SCAN_CH)], fp_buf, sem_in)

        def zero_init(i, carry):
            r = i // (FILL_CH // LANES)
            col = (i % (FILL_CH // LANES)) * LANES
            zero_buf[r, pl.ds(col, LANES)] = jnp.zeros((LANES,), jnp.float32)
            return carry

        lax.fori_loop(0, RB * FILL_CH // LANES, zero_init, 0)

        def scan_cond(state):
            ch = state[0]
            cnts = state[1:]
            cnt_min = cnts[0]
            for v in cnts[1:]:
                cnt_min = jnp.minimum(cnt_min, v)
            return jnp.logical_and(cnt_min < FILLER, ch < N_SCAN_CH)

        def scan_body(state):
            ch = state[0]
            cnts = list(state[1:])
            start = pl.multiple_of(ch * SCAN_CH, SCAN_CH)
            @pl.when(ch > 0)
            def _():
                pltpu.async_copy(
                    in_hbm.at[pl.ds(row0, RB), pl.ds(start, SCAN_CH)],
                    in_buf, sem_in)
                pltpu.async_copy(
                    fp_hbm.at[pl.ds(row0, RB), pl.ds(start, SCAN_CH)],
                    fp_buf, sem_in)
            pltpu.make_async_copy(
                in_hbm.at[pl.ds(row0, RB), pl.ds(start, SCAN_CH)],
                in_buf, sem_in).wait()
            pltpu.make_async_copy(
                fp_hbm.at[pl.ds(row0, RB), pl.ds(start, SCAN_CH)],
                fp_buf, sem_in).wait()
            for rr in range(RB):
                def vec_body(i, cnt, rr=rr):
                    vi = in_buf[rr, pl.ds(i * LANES, LANES)]
                    vf = fp_buf[rr, pl.ds(i * LANES, LANES)]
                    m = jnp.logical_and(vi > 0.0, vf > THRESH)
                    ones = jnp.where(m, jnp.float32(1.0), jnp.float32(0.0))
                    cs = plsc.cumsum(ones)
                    keep = jnp.logical_and(
                        m, (cnt.astype(jnp.float32) + cs)
                        <= jnp.float32(FILLER))
                    out_buf[rr, pl.ds(i * LANES, LANES)] = jnp.where(
                        keep, jnp.float32(1.0), jnp.float32(0.0))
                    return cnt + jnp.sum(ones).astype(jnp.int32)
                cnts[rr] = lax.fori_loop(
                    0, SCAN_CH // LANES, vec_body, cnts[rr])
            pltpu.async_copy(
                out_buf, out_hbm.at[pl.ds(row0, RB), pl.ds(start, SCAN_CH)],
                sem_out).wait()
            return (ch + 1, *cnts)

        end_state = lax.while_loop(
            scan_cond, scan_body, (0,) + (0,) * RB)
        ch_end = end_state[0]

        scan_end = pl.multiple_of(ch_end * SCAN_CH, SCAN_CH)
        need_align = jnp.logical_and(scan_end < COLS,
                                     scan_end % FILL_CH != 0)
        aligned = scan_end + jnp.where(need_align, SCAN_CH, 0)

        @pl.when(need_align)
        def _():
            pltpu.async_copy(
                zero_buf.at[slice(None), pl.ds(0, SCAN_CH)],
                out_hbm.at[pl.ds(row0, RB),
                           pl.ds(pl.multiple_of(scan_end, SCAN_CH), SCAN_CH)],
                sem_fill)

        def fill_fire(j, carry):
            st = pl.multiple_of(aligned + j * FILL_CH, FILL_CH)

            @pl.when(st < COLS)
            def _():
                pltpu.async_copy(
                    zero_buf,
                    out_hbm.at[pl.ds(row0, RB), pl.ds(st, FILL_CH)],
                    sem_fill)
            return carry

        lax.fori_loop(0, N_FILL, fill_fire, 0)

        @pl.when(need_align)
        def _():
            pltpu.make_async_copy(
                zero_buf.at[slice(None), pl.ds(0, SCAN_CH)],
                out_hbm.at[pl.ds(row0, RB),
                           pl.ds(pl.multiple_of(scan_end, SCAN_CH), SCAN_CH)],
                sem_fill).wait()

        def fill_drain(j, carry):
            st = pl.multiple_of(aligned + j * FILL_CH, FILL_CH)

            @pl.when(st < COLS)
            def _():
                pltpu.make_async_copy(
                    zero_buf,
                    out_hbm.at[pl.ds(row0, RB), pl.ds(st, FILL_CH)],
                    sem_fill).wait()
            return carry

        lax.fori_loop(0, N_FILL, fill_drain, 0)


@jax.jit
def _fake_profile_sc(inp, fp):
    mesh = plsc.VectorSubcoreMesh(
        core_axis_name="c", subcore_axis_name="s",
        num_cores=NC, num_subcores=NS)
    return pl.kernel(
        _sc_body,
        out_type=jax.ShapeDtypeStruct((ROWS, COLS), jnp.float32),
        mesh=mesh,
        scratch_types=[
            pltpu.VMEM((RB, SCAN_CH), jnp.float32),
            pltpu.VMEM((RB, SCAN_CH), jnp.float32),
            pltpu.VMEM((RB, SCAN_CH), jnp.float32),
            pltpu.VMEM((RB, FILL_CH), jnp.float32),
            pltpu.SemaphoreType.DMA,
            pltpu.SemaphoreType.DMA,
            pltpu.SemaphoreType.DMA,
        ],
        compiler_params=pltpu.CompilerParams(needs_layout_passes=False),
    )(inp, fp)


def kernel(input, fake_param):
    return _fake_profile_sc(input, fake_param)


# speedup vs baseline: 14.2947x; 1.0048x over previous
import jax
import jax.numpy as jnp
from jax import lax
from jax.experimental import pallas as pl
from jax.experimental.pallas import tpu as pltpu
from jax.experimental.pallas import tpu_sc as plsc

ROWS = 128
COLS = 32768
FILLER = 32

NC = 2
NS = 16
LANES = 16
RB = 8
N_BLOCKS = ROWS // RB

SCAN_CH = 512
N_SCAN_CH = COLS // SCAN_CH
FILL_CH = 1024
N_FILL = COLS // FILL_CH
THRESH = 0.5


def _sc_body(in_hbm, fp_hbm, out_hbm, in_buf, fp_buf, out_buf, zero_buf,
             sem_in, sem_out, sem_fill):
    c = lax.axis_index("c")
    s = lax.axis_index("s")
    blk = s
    row0 = blk * RB

    @pl.when(c == 0)
    def _owner():
        pltpu.async_copy(
            in_hbm.at[pl.ds(row0, RB), pl.ds(0, SCAN_CH)], in_buf, sem_in)
        pltpu.async_copy(
            fp_hbm.at[pl.ds(row0, RB), pl.ds(0, SCAN_CH)], fp_buf, sem_in)

        def zero_init(i, carry):
            r = i // (FILL_CH // LANES)
            col = (i % (FILL_CH // LANES)) * LANES
            zero_buf[r, pl.ds(col, LANES)] = jnp.zeros((LANES,), jnp.float32)
            return carry

        lax.fori_loop(0, RB * FILL_CH // LANES, zero_init, 0)

        def scan_cond(state):
            ch = state[0]
            cnts = state[1:]
            cnt_min = cnts[0]
            for v in cnts[1:]:
                cnt_min = jnp.minimum(cnt_min, v)
            return jnp.logical_and(cnt_min < FILLER, ch < N_SCAN_CH)

        def scan_body(state):
            ch = state[0]
            cnts = list(state[1:])
            start = pl.multiple_of(ch * SCAN_CH, SCAN_CH)
            @pl.when(ch > 0)
            def _():
                pltpu.async_copy(
                    in_hbm.at[pl.ds(row0, RB), pl.ds(start, SCAN_CH)],
                    in_buf, sem_in)
                pltpu.async_copy(
                    fp_hbm.at[pl.ds(row0, RB), pl.ds(start, SCAN_CH)],
                    fp_buf, sem_in)
            pltpu.make_async_copy(
                in_hbm.at[pl.ds(row0, RB), pl.ds(start, SCAN_CH)],
                in_buf, sem_in).wait()
            pltpu.make_async_copy(
                fp_hbm.at[pl.ds(row0, RB), pl.ds(start, SCAN_CH)],
                fp_buf, sem_in).wait()
            for rr in range(RB):
                def vec_body(i, cnt, rr=rr):
                    vi = in_buf[rr, pl.ds(i * LANES, LANES)]
                    vf = fp_buf[rr, pl.ds(i * LANES, LANES)]
                    m = jnp.logical_and(vi > 0.0, vf > THRESH)
                    ones = jnp.where(m, jnp.float32(1.0), jnp.float32(0.0))
                    cs = plsc.cumsum(ones)
                    keep = jnp.logical_and(
                        m, (cnt.astype(jnp.float32) + cs)
                        <= jnp.float32(FILLER))
                    out_buf[rr, pl.ds(i * LANES, LANES)] = jnp.where(
                        keep, jnp.float32(1.0), jnp.float32(0.0))
                    return cnt + jnp.sum(ones).astype(jnp.int32)
                cnts[rr] = lax.fori_loop(
                    0, SCAN_CH // LANES, vec_body, cnts[rr])
            pltpu.async_copy(
                out_buf, out_hbm.at[pl.ds(row0, RB), pl.ds(start, SCAN_CH)],
                sem_out).wait()
            return (ch + 1, *cnts)

        end_state = lax.while_loop(
            scan_cond, scan_body, (0,) + (0,) * RB)
        ch_end = end_state[0]

        scan_end = pl.multiple_of(ch_end * SCAN_CH, SCAN_CH)
        need_align = jnp.logical_and(scan_end < COLS,
                                     scan_end % FILL_CH != 0)
        aligned = scan_end + jnp.where(need_align, SCAN_CH, 0)

        @pl.when(need_align)
        def _():
            pltpu.async_copy(
                zero_buf.at[slice(None), pl.ds(0, SCAN_CH)],
                out_hbm.at[pl.ds(row0, RB),
                           pl.ds(pl.multiple_of(scan_end, SCAN_CH), SCAN_CH)],
                sem_fill)

        def fill_fire(j, carry):
            st = pl.multiple_of(aligned + j * FILL_CH, FILL_CH)

            @pl.when(st < COLS)
            def _():
                pltpu.async_copy(
                    zero_buf,
                    out_hbm.at[pl.ds(row0, RB), pl.ds(st, FILL_CH)],
                    sem_fill)
            return carry

        lax.fori_loop(0, N_FILL, fill_fire, 0)

        @pl.when(need_align)
        def _():
            pltpu.make_async_copy(
                zero_buf.at[slice(None), pl.ds(0, SCAN_CH)],
                out_hbm.at[pl.ds(row0, RB),
                           pl.ds(pl.multiple_of(scan_end, SCAN_CH), SCAN_CH)],
                sem_fill).wait()

        def fill_drain(j, carry):
            st = pl.multiple_of(aligned + j * FILL_CH, FILL_CH)

            @pl.when(st < COLS)
            def _():
                pltpu.make_async_copy(
                    zero_buf,
                    out_hbm.at[pl.ds(row0, RB), pl.ds(st, FILL_CH)],
                    sem_fill).wait()
            return carry

        lax.fori_loop(0, N_FILL, fill_drain, 0)


@jax.jit
def _fake_profile_sc(inp, fp):
    mesh = plsc.VectorSubcoreMesh(
        core_axis_name="c", subcore_axis_name="s",
        num_cores=NC, num_subcores=NS)
    return pl.kernel(
        _sc_body,
        out_type=jax.ShapeDtypeStruct((ROWS, COLS), jnp.float32),
        mesh=mesh,
        scratch_types=[
            pltpu.VMEM((RB, SCAN_CH), jnp.float32),
            pltpu.VMEM((RB, SCAN_CH), jnp.float32),
            pltpu.VMEM((RB, SCAN_CH), jnp.float32),
            pltpu.VMEM((RB, FILL_CH), jnp.float32),
            pltpu.SemaphoreType.DMA,
            pltpu.SemaphoreType.DMA,
            pltpu.SemaphoreType.DMA,
        ],
        compiler_params=pltpu.CompilerParams(needs_layout_passes=False),
    )(inp, fp)


def kernel(input, fake_param):
    return _fake_profile_sc(input, fake_param)


# speedup vs baseline: 18.3457x; 1.2834x over previous
import jax
import jax.numpy as jnp
from jax import lax
from jax.experimental import pallas as pl
from jax.experimental.pallas import tpu as pltpu
from jax.experimental.pallas import tpu_sc as plsc

ROWS = 128
COLS = 32768
FILLER = 32

NC = 2
NS = 16
LANES = 16
RB = 8
N_BLOCKS = ROWS // RB

SCAN_CH = 512
N_SCAN_CH = COLS // SCAN_CH
THRESH = 0.5

MEMSET_CH = 4096


def _memset_body(o_ref):
    o_ref[...] = jnp.zeros_like(o_ref)


def _tc_zeros():
    return pl.pallas_call(
        _memset_body,
        out_shape=jax.ShapeDtypeStruct((ROWS, COLS), jnp.float32),
        grid=(COLS // MEMSET_CH,),
        out_specs=pl.BlockSpec((ROWS, MEMSET_CH), lambda i: (0, i)),
    )()


def _sc_update(refs):
    in_hbm, fp_hbm, out_hbm = refs
    mesh = plsc.VectorSubcoreMesh(
        core_axis_name="c", subcore_axis_name="s",
        num_cores=NC, num_subcores=NS)

    @pl.core_map(
        mesh,
        compiler_params=pltpu.CompilerParams(needs_layout_passes=False))
    def _():
        c = lax.axis_index("c")
        s = lax.axis_index("s")
        row0 = s * RB

        def scoped(in_buf, fp_buf, out_buf, sem_in, sem_out):
            pltpu.async_copy(
                in_hbm.at[pl.ds(row0, RB), pl.ds(0, SCAN_CH)], in_buf,
                sem_in)
            pltpu.async_copy(
                fp_hbm.at[pl.ds(row0, RB), pl.ds(0, SCAN_CH)], fp_buf,
                sem_in)

            def scan_cond(state):
                ch = state[0]
                cnts = state[1:]
                cnt_min = cnts[0]
                for v in cnts[1:]:
                    cnt_min = jnp.minimum(cnt_min, v)
                return jnp.logical_and(cnt_min < FILLER, ch < N_SCAN_CH)

            def scan_body(state):
                ch = state[0]
                cnts = list(state[1:])
                start = pl.multiple_of(ch * SCAN_CH, SCAN_CH)

                @pl.when(ch > 0)
                def _():
                    pltpu.async_copy(
                        in_hbm.at[pl.ds(row0, RB), pl.ds(start, SCAN_CH)],
                        in_buf, sem_in)
                    pltpu.async_copy(
                        fp_hbm.at[pl.ds(row0, RB), pl.ds(start, SCAN_CH)],
                        fp_buf, sem_in)
                pltpu.make_async_copy(
                    in_hbm.at[pl.ds(row0, RB), pl.ds(start, SCAN_CH)],
                    in_buf, sem_in).wait()
                pltpu.make_async_copy(
                    fp_hbm.at[pl.ds(row0, RB), pl.ds(start, SCAN_CH)],
                    fp_buf, sem_in).wait()
                for rr in range(RB):
                    def vec_body(i, cnt, rr=rr):
                        vi = in_buf[rr, pl.ds(i * LANES, LANES)]
                        vf = fp_buf[rr, pl.ds(i * LANES, LANES)]
                        m = jnp.logical_and(vi > 0.0, vf > THRESH)
                        ones = jnp.where(m, jnp.float32(1.0),
                                         jnp.float32(0.0))
                        cs = plsc.cumsum(ones)
                        keep = jnp.logical_and(
                            m, (cnt.astype(jnp.float32) + cs)
                            <= jnp.float32(FILLER))
                        out_buf[rr, pl.ds(i * LANES, LANES)] = jnp.where(
                            keep, jnp.float32(1.0), jnp.float32(0.0))
                        return cnt + jnp.sum(ones).astype(jnp.int32)
                    cnts[rr] = lax.fori_loop(
                        0, SCAN_CH // LANES, vec_body, cnts[rr])
                pltpu.async_copy(
                    out_buf,
                    out_hbm.at[pl.ds(row0, RB), pl.ds(start, SCAN_CH)],
                    sem_out).wait()
                return (ch + 1, *cnts)

            lax.while_loop(scan_cond, scan_body, (0,) + (0,) * RB)

        @pl.when(c == 0)
        def _owner():
            pl.run_scoped(
                scoped,
                pltpu.VMEM((RB, SCAN_CH), jnp.float32),
                pltpu.VMEM((RB, SCAN_CH), jnp.float32),
                pltpu.VMEM((RB, SCAN_CH), jnp.float32),
                pltpu.SemaphoreType.DMA,
                pltpu.SemaphoreType.DMA,
            )


@jax.jit
def _fake_profile(inp, fp):
    zeros = _tc_zeros()
    _, _, out = pl.run_state(_sc_update)((inp, fp, zeros))
    return out


def kernel(input, fake_param):
    return _fake_profile(input, fake_param)


# speedup vs baseline: 19.4166x; 1.0584x over previous
import jax
import jax.numpy as jnp
from jax import lax
from jax.experimental import pallas as pl
from jax.experimental.pallas import tpu as pltpu
from jax.experimental.pallas import tpu_sc as plsc

ROWS = 128
COLS = 32768
FILLER = 32

NC = 2
NS = 16
LANES = 16
RB = 8
N_BLOCKS = ROWS // RB

SCAN_CH = 512
N_SCAN_CH = COLS // SCAN_CH
THRESH = 0.5

MEMSET_CH = 4096


def _memset_body(o_ref):
    o_ref[...] = jnp.zeros_like(o_ref)


def _tc_zeros():
    return pl.pallas_call(
        _memset_body,
        out_shape=jax.ShapeDtypeStruct((ROWS, COLS), jnp.float32),
        grid=(COLS // MEMSET_CH,),
        out_specs=pl.BlockSpec((ROWS, MEMSET_CH), lambda i: (0, i)),
    )()


def _sc_update(refs):
    in_hbm, fp_hbm, out_hbm = refs
    mesh = plsc.VectorSubcoreMesh(
        core_axis_name="c", subcore_axis_name="s",
        num_cores=1, num_subcores=NS)

    @pl.core_map(
        mesh,
        compiler_params=pltpu.CompilerParams(needs_layout_passes=False))
    def _():
        c = lax.axis_index("c")
        s = lax.axis_index("s")
        row0 = s * RB

        def scoped(in_buf, fp_buf, out_buf, sem_in, sem_out):
            pltpu.async_copy(
                in_hbm.at[pl.ds(row0, RB), pl.ds(0, SCAN_CH)], in_buf,
                sem_in)
            pltpu.async_copy(
                fp_hbm.at[pl.ds(row0, RB), pl.ds(0, SCAN_CH)], fp_buf,
                sem_in)

            def scan_cond(state):
                ch = state[0]
                cnts = state[1:]
                cnt_min = cnts[0]
                for v in cnts[1:]:
                    cnt_min = jnp.minimum(cnt_min, v)
                return jnp.logical_and(cnt_min < FILLER, ch < N_SCAN_CH)

            def scan_body(state):
                ch = state[0]
                cnts = list(state[1:])
                start = pl.multiple_of(ch * SCAN_CH, SCAN_CH)

                @pl.when(ch > 0)
                def _():
                    pltpu.async_copy(
                        in_hbm.at[pl.ds(row0, RB), pl.ds(start, SCAN_CH)],
                        in_buf, sem_in)
                    pltpu.async_copy(
                        fp_hbm.at[pl.ds(row0, RB), pl.ds(start, SCAN_CH)],
                        fp_buf, sem_in)
                pltpu.make_async_copy(
                    in_hbm.at[pl.ds(row0, RB), pl.ds(start, SCAN_CH)],
                    in_buf, sem_in).wait()
                pltpu.make_async_copy(
                    fp_hbm.at[pl.ds(row0, RB), pl.ds(start, SCAN_CH)],
                    fp_buf, sem_in).wait()
                for rr in range(RB):
                    def vec_body(i, cnt, rr=rr):
                        vi = in_buf[rr, pl.ds(i * LANES, LANES)]
                        vf = fp_buf[rr, pl.ds(i * LANES, LANES)]
                        m = jnp.logical_and(vi > 0.0, vf > THRESH)
                        ones = jnp.where(m, jnp.float32(1.0),
                                         jnp.float32(0.0))
                        cs = plsc.cumsum(ones)
                        keep = jnp.logical_and(
                            m, (cnt.astype(jnp.float32) + cs)
                            <= jnp.float32(FILLER))
                        out_buf[rr, pl.ds(i * LANES, LANES)] = jnp.where(
                            keep, jnp.float32(1.0), jnp.float32(0.0))
                        return cnt + jnp.sum(ones).astype(jnp.int32)
                    cnts[rr] = lax.fori_loop(
                        0, SCAN_CH // LANES, vec_body, cnts[rr])
                pltpu.async_copy(
                    out_buf,
                    out_hbm.at[pl.ds(row0, RB), pl.ds(start, SCAN_CH)],
                    sem_out).wait()
                return (ch + 1, *cnts)

            lax.while_loop(scan_cond, scan_body, (0,) + (0,) * RB)

        @pl.when(c == 0)
        def _owner():
            pl.run_scoped(
                scoped,
                pltpu.VMEM((RB, SCAN_CH), jnp.float32),
                pltpu.VMEM((RB, SCAN_CH), jnp.float32),
                pltpu.VMEM((RB, SCAN_CH), jnp.float32),
                pltpu.SemaphoreType.DMA,
                pltpu.SemaphoreType.DMA,
            )


@jax.jit
def _fake_profile(inp, fp):
    zeros = _tc_zeros()
    _, _, out = pl.run_state(_sc_update)((inp, fp, zeros))
    return out


def kernel(input, fake_param):
    return _fake_profile(input, fake_param)
